# baseline (speedup 1.0000x reference)
import functools

import jax
import jax.numpy as jnp
from jax.experimental import pallas as pl
from jax.experimental.pallas import tpu as pltpu


def _fused_kernel(a_ref, x_ref, w1_ref, b1_ref, w2_ref, b2_ref, p_ref,
                  o_ref, h1_ref, h4_ref, *, bm):
    g = pl.program_id(0)
    m = pl.program_id(1)

    @pl.when((g == 0) & (m == 0))
    def _fc1():
        h = jax.lax.dot_general(
            x_ref[...], w1_ref[...], (((1,), (0,)), ((), ())),
            preferred_element_type=jnp.float32,
        )
        h1_ref[...] = (h + b1_ref[...]).astype(h1_ref.dtype)

    a = a_ref[0].astype(jnp.bfloat16)

    @pl.when(g == 0)
    def _pass_a():
        r = jax.lax.dot_general(
            a, h1_ref[...], (((1,), (0,)), ((), ())),
            preferred_element_type=jnp.float32,
        )
        p = p_ref[0, 0]
        r = jnp.maximum(r, 0.0) + p * jnp.minimum(r, 0.0)
        r = jax.lax.dot_general(
            r.astype(jnp.bfloat16), w2_ref[...], (((1,), (0,)), ((), ())),
            preferred_element_type=jnp.float32,
        ) + b2_ref[...]
        h4_ref[pl.ds(m * bm, bm), :] = r.astype(h4_ref.dtype)

    @pl.when(g == 1)
    def _pass_b():
        o_ref[...] = jax.lax.dot_general(
            a, h4_ref[...], (((1,), (0,)), ((), ())),
            preferred_element_type=jnp.float32,
        ).astype(o_ref.dtype)


def _pick(n, candidates):
    for c in candidates:
        if n % c == 0:
            return c
    return n


def kernel(x, adj, W1, b1, W2, b2, prelu_a):
    _, n, f = x.shape
    d = W1.shape[0]
    xs = x.reshape(n, f)
    w1t = W1.T
    w2t = W2.T.astype(jnp.bfloat16)
    b1r = b1.reshape(1, d)
    b2r = b2.reshape(1, d)
    pa = prelu_a.reshape(1, 1)

    bm = _pick(n, (200, 400, 100, 8))

    out = pl.pallas_call(
        functools.partial(_fused_kernel, bm=bm),
        grid=(2, n // bm),
        in_specs=[
            pl.BlockSpec((1, bm, n), lambda g, m: (0, m, 0)),
            pl.BlockSpec((n, f), lambda g, m: (0, 0)),
            pl.BlockSpec((f, d), lambda g, m: (0, 0)),
            pl.BlockSpec((1, d), lambda g, m: (0, 0)),
            pl.BlockSpec((d, d), lambda g, m: (0, 0)),
            pl.BlockSpec((1, d), lambda g, m: (0, 0)),
            pl.BlockSpec((1, 1), lambda g, m: (0, 0)),
        ],
        out_specs=pl.BlockSpec(
            (bm, d), lambda g, m: (jnp.where(g == 1, m, 0), 0)),
        out_shape=jax.ShapeDtypeStruct((n, d), jnp.float32),
        scratch_shapes=[
            pltpu.VMEM((n, d), jnp.bfloat16),
            pltpu.VMEM((n, d), jnp.bfloat16),
        ],
        compiler_params=pltpu.CompilerParams(
            dimension_semantics=("arbitrary", "arbitrary")),
    )(adj, xs, w1t, b1r, w2t, b2r, pa)
    return out.reshape(1, n, d)


# speedup vs baseline: 1.0965x; 1.0965x over previous
import functools

import jax
import jax.numpy as jnp
from jax.experimental import pallas as pl
from jax.experimental.pallas import tpu as pltpu


def _fused_kernel(a_ref, x_ref, w1_ref, b1_ref, w2_ref, b2_ref, p_ref,
                  o_ref, h1_ref, h4_ref, *, bm):
    g = pl.program_id(0)
    m = pl.program_id(1)

    @pl.when((g == 0) & (m == 0))
    def _fc1():
        h = jax.lax.dot_general(
            x_ref[...], w1_ref[...], (((1,), (0,)), ((), ())),
            preferred_element_type=jnp.float32,
        )
        h1_ref[...] = (h + b1_ref[...]).astype(h1_ref.dtype)

    a = a_ref[0].astype(jnp.bfloat16)

    @pl.when(g == 0)
    def _pass_a():
        r = jax.lax.dot_general(
            a, h1_ref[...], (((1,), (0,)), ((), ())),
            preferred_element_type=jnp.float32,
        )
        p = p_ref[0, 0]
        r = jnp.maximum(r, 0.0) + p * jnp.minimum(r, 0.0)
        r = jax.lax.dot_general(
            r.astype(jnp.bfloat16), w2_ref[...], (((1,), (0,)), ((), ())),
            preferred_element_type=jnp.float32,
        ) + b2_ref[...]
        h4_ref[pl.ds(m * bm, bm), :] = r.astype(h4_ref.dtype)

    @pl.when(g == 1)
    def _pass_b():
        o_ref[...] = jax.lax.dot_general(
            a, h4_ref[...], (((1,), (0,)), ((), ())),
            preferred_element_type=jnp.float32,
        ).astype(o_ref.dtype)


def _pick(n, candidates):
    for c in candidates:
        if n % c == 0:
            return c
    return n


def kernel(x, adj, W1, b1, W2, b2, prelu_a):
    _, n, f = x.shape
    d = W1.shape[0]
    xs = x.reshape(n, f)
    w1t = W1.T
    w2t = W2.T.astype(jnp.bfloat16)
    b1r = b1.reshape(1, d)
    b2r = b2.reshape(1, d)
    pa = prelu_a.reshape(1, 1)

    bm = _pick(n, (400, 200, 100, 8))

    out = pl.pallas_call(
        functools.partial(_fused_kernel, bm=bm),
        grid=(2, n // bm),
        in_specs=[
            pl.BlockSpec((1, bm, n), lambda g, m: (0, m, 0)),
            pl.BlockSpec((n, f), lambda g, m: (0, 0)),
            pl.BlockSpec((f, d), lambda g, m: (0, 0)),
            pl.BlockSpec((1, d), lambda g, m: (0, 0)),
            pl.BlockSpec((d, d), lambda g, m: (0, 0)),
            pl.BlockSpec((1, d), lambda g, m: (0, 0)),
            pl.BlockSpec((1, 1), lambda g, m: (0, 0)),
        ],
        out_specs=pl.BlockSpec(
            (bm, d), lambda g, m: (jnp.where(g == 1, m, 0), 0)),
        out_shape=jax.ShapeDtypeStruct((n, d), jnp.float32),
        scratch_shapes=[
            pltpu.VMEM((n, d), jnp.bfloat16),
            pltpu.VMEM((n, d), jnp.bfloat16),
        ],
        compiler_params=pltpu.CompilerParams(
            dimension_semantics=("arbitrary", "arbitrary")),
    )(adj, xs, w1t, b1r, w2t, b2r, pa)
    return out.reshape(1, n, d)
